# baseline (speedup 1.0000x reference)
import functools
import math

import jax
import jax.numpy as jnp
from jax import lax
from jax.experimental import pallas as pl
from jax.experimental.pallas import tpu as pltpu
from jax.experimental.pallas import tpu_sc as plsc

D_MODEL = 64
SCALE = math.sqrt(D_MODEL)
CHUNK = 128
NC, NS, LANES = 2, 16, 16
NW = NC * NS


def _emb_body(chunks_per_w, table_hbm, idx_hbm, out_hbm,
              idx_v, b0, b1, b2, s0, s1, s2, o0, o1, o2):
    bufs = (b0, b1, b2)
    sems = (s0, s1, s2)
    osems = (o0, o1, o2)
    n_pairs = chunks_per_w // 2
    wid = lax.axis_index("s") * NC + lax.axis_index("c")
    rows_per_w = chunks_per_w * CHUNK
    base = wid * rows_per_w
    pltpu.sync_copy(idx_hbm.at[pl.ds(base, rows_per_w)], idx_v)

    def fire(p, k):
        for h in range(2):
            idx_sl = idx_v.at[pl.ds((2 * p + h) * CHUNK, CHUNK)]
            pltpu.async_copy(table_hbm.at[idx_sl],
                             bufs[k].at[pl.ds(h * CHUNK, CHUNK)], sems[k])

    def drain(k):
        for h in range(2):
            pltpu.make_async_copy(table_hbm.at[pl.ds(0, CHUNK)],
                                  bufs[k].at[pl.ds(h * CHUNK, CHUNK)],
                                  sems[k]).wait()

    def wait_out(k):
        pltpu.make_async_copy(bufs[k], out_hbm.at[pl.ds(0, 2 * CHUNK)],
                              osems[k]).wait()

    def process(p, k):
        drain(k)
        kn = (k + 1) % 3

        @pl.when(p + 1 < n_pairs)
        def _():
            @pl.when(p >= 2)
            def _():
                wait_out(kn)

            fire(p + 1, kn)

        def row_body(r, _):
            for j in range(D_MODEL // LANES):
                sl = pl.ds(j * LANES, LANES)
                bufs[k][r, sl] = bufs[k][r, sl] * SCALE
            return 0

        lax.fori_loop(0, 2 * CHUNK, row_body, 0, unroll=8)
        pltpu.async_copy(bufs[k],
                         out_hbm.at[pl.ds(base + 2 * p * CHUNK, 2 * CHUNK)],
                         osems[k])

    fire(0, 0)

    def step(i, _):
        for k in range(3):
            process(i * 3 + k, k)
        return 0

    lax.fori_loop(0, n_pairs // 3, step, 0)
    for r in range(n_pairs % 3):
        process(n_pairs - (n_pairs % 3) + r, r)
    for k in range(min(3, n_pairs)):
        wait_out(k)


@jax.jit
def _emb_lookup(lut, idx):
    n_rows = idx.shape[0]
    chunks_per_w = n_rows // (NW * CHUNK)
    mesh = plsc.VectorSubcoreMesh(core_axis_name="c", subcore_axis_name="s")
    k = pl.kernel(
        functools.partial(_emb_body, chunks_per_w),
        mesh=mesh,
        out_type=jax.ShapeDtypeStruct((n_rows, D_MODEL), jnp.float32),
        scratch_types=[
            pltpu.VMEM((chunks_per_w * CHUNK,), jnp.int32),
            pltpu.VMEM((2 * CHUNK, D_MODEL), jnp.float32),
            pltpu.VMEM((2 * CHUNK, D_MODEL), jnp.float32),
            pltpu.VMEM((2 * CHUNK, D_MODEL), jnp.float32),
            pltpu.SemaphoreType.DMA, pltpu.SemaphoreType.DMA,
            pltpu.SemaphoreType.DMA, pltpu.SemaphoreType.DMA,
            pltpu.SemaphoreType.DMA, pltpu.SemaphoreType.DMA,
        ],
        compiler_params=pltpu.CompilerParams(use_tc_tiling_on_sc=False),
    )
    return k(lut, idx)


def kernel(x, lut):
    b, s = x.shape
    idx = x.reshape(-1).astype(jnp.int32)
    out = _emb_lookup(lut, idx)
    return out.reshape(b, s, D_MODEL)
